# Initial kernel scaffold; baseline (speedup 1.0000x reference)
import jax
import jax.numpy as jnp
from jax.experimental import pallas as pl


def kernel(x, edge_index, batch_idx, emb_table, ln1_g, ln1_b, W1, a_src1, a_dst1, b1, W2, a_src2, a_dst2, b2, gW1, gb1, gW2, gb2, ln2_g, ln2_b, fW1, fb1, fW2, fb2):
    raise NotImplementedError("write your pallas kernel here")


# speedup vs baseline: 29.4330x; 29.4330x over previous
import functools

import jax
import jax.numpy as jnp
from jax import lax
from jax.experimental import pallas as pl
from jax.experimental.pallas import tpu as pltpu
from jax.experimental.pallas import tpu_sc as plsc

N = 10000
E = 320000
V = 5001
D = 128
HID = 128
HEADS = 2
B = 64

NP = 10240
NSUB = 16
NCORE = 2
EC = 128
CHUNKS = 157
EPS = NP // NSUB
E_PAD = NSUB * CHUNKS * EC
GR = 320 // 4

_mesh = plsc.VectorSubcoreMesh(
    core_axis_name="c", subcore_axis_name="s", num_cores=NCORE, num_subcores=NSUB)


def _leaky(x, s):
    return jnp.where(x > 0, x, s * x)


def _emb_body(tab_hbm, idx_hbm, out_hbm, idx_v, rows_v, sem):
    c = lax.axis_index("c")
    s = lax.axis_index("s")
    wid = s * NCORE + c
    base = wid * (NP // (NCORE * NSUB))

    def chunk(i, _):
        off = base + i * GR
        pltpu.sync_copy(idx_hbm.at[pl.ds(off, GR)], idx_v)
        pltpu.async_copy(tab_hbm.at[idx_v], rows_v, sem).wait()
        pltpu.sync_copy(rows_v, out_hbm.at[pl.ds(off, GR)])
        return 0

    lax.fori_loop(0, (NP // (NCORE * NSUB)) // GR, chunk, 0)


_emb_gather = pl.kernel(
    _emb_body,
    out_type=jax.ShapeDtypeStruct((NP, D), jnp.float32),
    mesh=_mesh,
    scratch_types=[
        pltpu.VMEM((GR,), jnp.int32),
        pltpu.VMEM((GR, D), jnp.float32),
        pltpu.SemaphoreType.DMA,
    ],
)


def _edge_body(src_hbm, dst_hbm, hcat_hbm, acat_hbm, bcat_hbm,
               znd_hbm, zn_hbm,
               numcat_hbm, dencat_hbm,
               sidx, didx_l, didx_g, av, bv, wv, wpad, rows, acc, den_sh,
               sem_r, sem_a, sem_b):
    c = lax.axis_index("c")
    s = lax.axis_index("s")
    r0 = s * EPS
    coff = c * NP
    eoff = c * E_PAD

    pltpu.sync_copy(znd_hbm.at[pl.ds(r0, EPS)], acc.at[pl.ds(r0, EPS)])
    pltpu.sync_copy(zn_hbm.at[pl.ds(r0, EPS)], den_sh.at[pl.ds(r0, EPS)])
    plsc.subcore_barrier()

    ebase = s * (CHUNKS * EC)

    def chunk(i, _):
        off = ebase + i * EC
        pltpu.sync_copy(src_hbm.at[pl.ds(eoff + off, EC)], sidx)
        pltpu.sync_copy(dst_hbm.at[pl.ds(eoff + off, EC)], didx_g)
        pltpu.sync_copy(dst_hbm.at[pl.ds(off, EC)], didx_l)
        cp_r = pltpu.async_copy(hcat_hbm.at[sidx], rows, sem_r)
        cp_a = pltpu.async_copy(acat_hbm.at[sidx], av, sem_a)
        cp_b = pltpu.async_copy(bcat_hbm.at[didx_g], bv, sem_b)
        cp_a.wait()
        cp_b.wait()
        for j in range(EC // 16):
            v = av[pl.ds(j * 16, 16)] + bv[pl.ds(j * 16, 16)]
            w16 = jnp.exp(_leaky(v, 0.2))
            wv[pl.ds(j * 16, 16)] = w16
            wpad[pl.ds(j * 16, 16)] = w16
        cp_r.wait()

        def scale(e, _):
            ws = wpad[pl.ds(e, 16)][0]
            for f in range(D // 16):
                rows[e, pl.ds(f * 16, 16)] = rows[e, pl.ds(f * 16, 16)] * ws
            return 0

        lax.fori_loop(0, EC, scale, 0)
        pltpu.sync_copy(rows, acc.at[didx_l], add=True)
        pltpu.sync_copy(wv, den_sh.at[didx_l], add=True)
        return 0

    lax.fori_loop(0, CHUNKS, chunk, 0)
    plsc.subcore_barrier()
    pltpu.sync_copy(acc.at[pl.ds(r0, EPS)], numcat_hbm.at[pl.ds(coff + r0, EPS)])
    pltpu.sync_copy(den_sh.at[pl.ds(r0, EPS)], dencat_hbm.at[pl.ds(coff + r0, EPS)])


_edge_sc = pl.kernel(
    _edge_body,
    out_type=(
        jax.ShapeDtypeStruct((2 * NP, HID), jnp.float32),
        jax.ShapeDtypeStruct((2 * NP,), jnp.float32),
    ),
    mesh=_mesh,
    scratch_types=[
        pltpu.VMEM((EC,), jnp.int32),
        pltpu.VMEM((EC,), jnp.int32),
        pltpu.VMEM((EC,), jnp.int32),
        pltpu.VMEM((EC,), jnp.float32),
        pltpu.VMEM((EC,), jnp.float32),
        pltpu.VMEM((EC,), jnp.float32),
        pltpu.VMEM((EC + 16,), jnp.float32),
        pltpu.VMEM((EC, HID), jnp.float32),
        pltpu.VMEM_SHARED((NP, HID), jnp.float32),
        pltpu.VMEM_SHARED((NP,), jnp.float32),
        pltpu.SemaphoreType.DMA,
        pltpu.SemaphoreType.DMA,
        pltpu.SemaphoreType.DMA,
    ],
)


def _tca_body(xe_ref, g_ref, b_ref, w_ref, as_ref, ad_ref,
              h0_ref, h1_ref, av0_ref, av1_ref, bv0_ref, bv1_ref):
    xb = xe_ref[...]
    mu = jnp.mean(xb, axis=1, keepdims=True)
    var = jnp.mean((xb - mu) ** 2, axis=1, keepdims=True)
    ln = (xb - mu) * lax.rsqrt(var + 1e-5) * g_ref[...] + b_ref[...]
    h = jnp.dot(ln, w_ref[...], preferred_element_type=jnp.float32)
    h0 = h[:, :HID]
    h1 = h[:, HID:]
    h0_ref[...] = h0
    h1_ref[...] = h1
    av0_ref[...] = jnp.sum(h0 * as_ref[0:1, :], axis=1, keepdims=True)
    av1_ref[...] = jnp.sum(h1 * as_ref[1:2, :], axis=1, keepdims=True)
    bv0_ref[...] = jnp.sum(h0 * ad_ref[0:1, :], axis=1, keepdims=True)
    bv1_ref[...] = jnp.sum(h1 * ad_ref[1:2, :], axis=1, keepdims=True)


_TCA_BN = 256


def _tc_a(xe, g, b, W, a_s, a_d):
    grid = (NP // _TCA_BN,)
    row = lambda i: (i, 0)
    fixed = lambda i: (0, 0)
    return pl.pallas_call(
        _tca_body,
        grid=grid,
        in_specs=[
            pl.BlockSpec((_TCA_BN, D), row),
            pl.BlockSpec((1, D), fixed),
            pl.BlockSpec((1, D), fixed),
            pl.BlockSpec((D, HEADS * HID), fixed),
            pl.BlockSpec((HEADS, HID), fixed),
            pl.BlockSpec((HEADS, HID), fixed),
        ],
        out_specs=[
            pl.BlockSpec((_TCA_BN, HID), row),
            pl.BlockSpec((_TCA_BN, HID), row),
            pl.BlockSpec((_TCA_BN, 1), row),
            pl.BlockSpec((_TCA_BN, 1), row),
            pl.BlockSpec((_TCA_BN, 1), row),
            pl.BlockSpec((_TCA_BN, 1), row),
        ],
        out_shape=[
            jax.ShapeDtypeStruct((NP, HID), jnp.float32),
            jax.ShapeDtypeStruct((NP, HID), jnp.float32),
            jax.ShapeDtypeStruct((NP, 1), jnp.float32),
            jax.ShapeDtypeStruct((NP, 1), jnp.float32),
            jax.ShapeDtypeStruct((NP, 1), jnp.float32),
            jax.ShapeDtypeStruct((NP, 1), jnp.float32),
        ],
    )(xe, g.reshape(1, D), b.reshape(1, D), W, a_s.reshape(HEADS, HID),
      a_d.reshape(HEADS, HID))


def _tcb_body(n0_ref, n1_ref, d0_ref, d1_ref, h0_ref, h1_ref,
              av0_ref, av1_ref, bv0_ref, bv1_ref, b1_ref, w2_ref, as_ref, ad_ref,
              o0_ref, o1_ref, av20_ref, av21_ref, bv20_ref, bv21_ref):
    ws0 = jnp.exp(_leaky(av0_ref[...] + bv0_ref[...], 0.2))
    ws1 = jnp.exp(_leaky(av1_ref[...] + bv1_ref[...], 0.2))
    o0 = (n0_ref[...] + ws0 * h0_ref[...]) / (d0_ref[...] + ws0 + 1e-16)
    o1 = (n1_ref[...] + ws1 * h1_ref[...]) / (d1_ref[...] + ws1 + 1e-16)
    gcat = jnp.concatenate([o0, o1], axis=1) + b1_ref[...]
    gcat = _leaky(gcat, 0.05)
    h2 = jnp.dot(gcat, w2_ref[...], preferred_element_type=jnp.float32)
    h20 = h2[:, :HID]
    h21 = h2[:, HID:]
    o0_ref[...] = h20
    o1_ref[...] = h21
    av20_ref[...] = jnp.sum(h20 * as_ref[0:1, :], axis=1, keepdims=True)
    av21_ref[...] = jnp.sum(h21 * as_ref[1:2, :], axis=1, keepdims=True)
    bv20_ref[...] = jnp.sum(h20 * ad_ref[0:1, :], axis=1, keepdims=True)
    bv21_ref[...] = jnp.sum(h21 * ad_ref[1:2, :], axis=1, keepdims=True)


def _tc_b(n0, n1, d0, d1, h0, h1, av0, av1, bv0, bv1, b1, W2, a_s, a_d):
    grid = (NP // _TCA_BN,)
    row = lambda i: (i, 0)
    fixed = lambda i: (0, 0)
    col = pl.BlockSpec((_TCA_BN, 1), row)
    mat = pl.BlockSpec((_TCA_BN, HID), row)
    return pl.pallas_call(
        _tcb_body,
        grid=grid,
        in_specs=[mat, mat, col, col, mat, mat, col, col, col, col,
                  pl.BlockSpec((1, HEADS * HID), fixed),
                  pl.BlockSpec((HEADS * HID, HEADS * HID), fixed),
                  pl.BlockSpec((HEADS, HID), fixed),
                  pl.BlockSpec((HEADS, HID), fixed)],
        out_specs=[mat, mat, col, col, col, col],
        out_shape=[
            jax.ShapeDtypeStruct((NP, HID), jnp.float32),
            jax.ShapeDtypeStruct((NP, HID), jnp.float32),
            jax.ShapeDtypeStruct((NP, 1), jnp.float32),
            jax.ShapeDtypeStruct((NP, 1), jnp.float32),
            jax.ShapeDtypeStruct((NP, 1), jnp.float32),
            jax.ShapeDtypeStruct((NP, 1), jnp.float32),
        ],
    )(n0, n1, d0, d1, h0, h1, av0, av1, bv0, bv1,
      b1.reshape(1, HEADS * HID), W2, a_s.reshape(HEADS, HID), a_d.reshape(HEADS, HID))


_TCC_BN = 200
_TCC_STEPS = N // _TCC_BN


def _tcc_body(n0_ref, n1_ref, d0_ref, d1_ref, h0_ref, h1_ref,
              av0_ref, av1_ref, bv0_ref, bv1_ref, bidx_ref,
              b2_ref, gw1_ref, gb1_ref, gw2_ref, gb2_ref,
              ln2g_ref, ln2b_ref, fw1_ref, fb1_ref, fw2_ref, fb2_ref,
              oc_ref, hid_ref, s_acc):
    i = pl.program_id(0)

    ws0 = jnp.exp(_leaky(av0_ref[...] + bv0_ref[...], 0.2))
    ws1 = jnp.exp(_leaky(av1_ref[...] + bv1_ref[...], 0.2))
    o0 = (n0_ref[...] + ws0 * h0_ref[...]) / (d0_ref[...] + ws0 + 1e-16)
    o1 = (n1_ref[...] + ws1 * h1_ref[...]) / (d1_ref[...] + ws1 + 1e-16)
    oc = jnp.concatenate([o0, o1], axis=1) + b2_ref[...]
    oc_ref[...] = oc

    gate = _leaky(jnp.dot(oc, gw1_ref[...], preferred_element_type=jnp.float32)
                  + gb1_ref[...], 0.05)
    gate = jnp.dot(gate, gw2_ref[...], preferred_element_type=jnp.float32) + gb2_ref[...]
    e = jnp.exp(gate)

    ids = lax.broadcasted_iota(jnp.int32, (1, B), 1)
    m = (bidx_ref[...] == ids).astype(jnp.float32)
    xcat = jnp.concatenate([e * oc, jnp.broadcast_to(e, (_TCC_BN, HID))], axis=1)
    part = lax.dot_general(m, xcat, (((0,), (0,)), ((), ())),
                           preferred_element_type=jnp.float32)

    @pl.when(i == 0)
    def _():
        s_acc[...] = jnp.zeros_like(s_acc)

    s_acc[...] += part

    @pl.when(i == _TCC_STEPS - 1)
    def _():
        s = s_acc[...]
        hidden = s[:, :HEADS * HID] / (s[:, HEADS * HID:HEADS * HID + 1] + 1e-16)
        mu = jnp.mean(hidden, axis=1, keepdims=True)
        var = jnp.mean((hidden - mu) ** 2, axis=1, keepdims=True)
        hidden = (hidden - mu) * lax.rsqrt(var + 1e-5) * ln2g_ref[...] + ln2b_ref[...]
        hidden = _leaky(jnp.dot(hidden, fw1_ref[...],
                                preferred_element_type=jnp.float32) + fb1_ref[...], 0.05)
        hid_ref[...] = jnp.dot(hidden, fw2_ref[...],
                               preferred_element_type=jnp.float32) + fb2_ref[...]


def _tc_c(n0, n1, d0, d1, h0, h1, av0, av1, bv0, bv1, bidx,
          b2, gW1, gb1, gW2, gb2, ln2g, ln2b, fW1, fb1, fW2, fb2):
    grid = (_TCC_STEPS,)
    row = lambda i: (i, 0)
    fixed = lambda i: (0, 0)
    col = pl.BlockSpec((_TCC_BN, 1), row)
    mat = pl.BlockSpec((_TCC_BN, HID), row)
    return pl.pallas_call(
        _tcc_body,
        grid=grid,
        in_specs=[mat, mat, col, col, mat, mat, col, col, col, col,
                  pl.BlockSpec((_TCC_BN, 1), row),
                  pl.BlockSpec((1, HEADS * HID), fixed),
                  pl.BlockSpec((HEADS * HID, HID), fixed),
                  pl.BlockSpec((1, HID), fixed),
                  pl.BlockSpec((HID, 1), fixed),
                  pl.BlockSpec((1, 1), fixed),
                  pl.BlockSpec((1, HEADS * HID), fixed),
                  pl.BlockSpec((1, HEADS * HID), fixed),
                  pl.BlockSpec((HEADS * HID, HEADS * HID), fixed),
                  pl.BlockSpec((1, HEADS * HID), fixed),
                  pl.BlockSpec((HEADS * HID, HID), fixed),
                  pl.BlockSpec((1, HID), fixed)],
        out_specs=[
            pl.BlockSpec((_TCC_BN, HEADS * HID), row),
            pl.BlockSpec((B, HID), fixed),
        ],
        out_shape=[
            jax.ShapeDtypeStruct((N, HEADS * HID), jnp.float32),
            jax.ShapeDtypeStruct((B, HID), jnp.float32),
        ],
        scratch_shapes=[pltpu.VMEM((B, HEADS * HID + HID), jnp.float32)],
    )(n0, n1, d0, d1, h0, h1, av0, av1, bv0, bv1, bidx,
      b2.reshape(1, HEADS * HID), gW1, gb1.reshape(1, HID), gW2,
      gb2.reshape(1, 1), ln2g.reshape(1, HEADS * HID), ln2b.reshape(1, HEADS * HID),
      fW1, fb1.reshape(1, HEADS * HID), fW2, fb2.reshape(1, HID))


def kernel(x, edge_index, batch_idx, emb_table, ln1_g, ln1_b, W1, a_src1, a_dst1, b1,
           W2, a_src2, a_dst2, b2, gW1, gb1, gW2, gb2, ln2_g, ln2_b, fW1, fb1, fW2, fb2):
    pad = E_PAD - E
    src = jnp.concatenate([edge_index[0].astype(jnp.int32),
                           jnp.zeros((pad,), jnp.int32)])
    dst = jnp.concatenate([edge_index[1].astype(jnp.int32),
                           jnp.full((pad,), NP - 1, jnp.int32)])
    xp = jnp.concatenate([x[:, 0].astype(jnp.int32), jnp.zeros((NP - N,), jnp.int32)])
    znd = jnp.zeros((NP, HID), jnp.float32)
    zn = jnp.zeros((NP,), jnp.float32)

    xe = _emb_gather(emb_table, xp)
    h0, h1, av0, av1, bv0, bv1 = _tc_a(xe, ln1_g, ln1_b, W1, a_src1, a_dst1)
    srccat = jnp.concatenate([src, src + NP])
    dstcat = jnp.concatenate([dst, dst + NP])
    ncat, dcat = _edge_sc(srccat, dstcat,
                          jnp.concatenate([h0, h1]),
                          jnp.concatenate([av0.reshape(NP), av1.reshape(NP)]),
                          jnp.concatenate([bv0.reshape(NP), bv1.reshape(NP)]),
                          znd, zn)
    h20, h21, av20, av21, bv20, bv21 = _tc_b(
        ncat[:NP], ncat[NP:], dcat[:NP].reshape(NP, 1), dcat[NP:].reshape(NP, 1),
        h0, h1, av0, av1, bv0, bv1, b1, W2, a_src2, a_dst2)
    ncat, dcat = _edge_sc(srccat, dstcat,
                          jnp.concatenate([h20, h21]),
                          jnp.concatenate([av20.reshape(NP), av21.reshape(NP)]),
                          jnp.concatenate([bv20.reshape(NP), bv21.reshape(NP)]),
                          znd, zn)
    out_conv, hidden = _tc_c(
        ncat[:NP], ncat[NP:], dcat[:NP].reshape(NP, 1), dcat[NP:].reshape(NP, 1),
        h20, h21, av20, av21, bv20, bv21, batch_idx.astype(jnp.int32).reshape(N, 1),
        b2, gW1, gb1, gW2, gb2, ln2_g, ln2_b, fW1, fb1, fW2, fb2)
    return out_conv, hidden


# speedup vs baseline: 36.1083x; 1.2268x over previous
import functools

import jax
import jax.numpy as jnp
from jax import lax
from jax.experimental import pallas as pl
from jax.experimental.pallas import tpu as pltpu
from jax.experimental.pallas import tpu_sc as plsc

N = 10000
E = 320000
V = 5001
D = 128
HID = 128
HEADS = 2
B = 64

NP = 10240
NSUB = 16
NCORE = 2
EC = 128
CHUNKS = 160
EPS = NP // NSUB
E_PAD = NSUB * CHUNKS * EC
GR = 320 // 4

_mesh = plsc.VectorSubcoreMesh(
    core_axis_name="c", subcore_axis_name="s", num_cores=NCORE, num_subcores=NSUB)


def _leaky(x, s):
    return jnp.where(x > 0, x, s * x)


def _emb_body(tab_hbm, idx_hbm, out_hbm, idx_v, rows_v, sem):
    c = lax.axis_index("c")
    s = lax.axis_index("s")
    wid = s * NCORE + c
    base = wid * (NP // (NCORE * NSUB))

    def chunk(i, _):
        off = base + i * GR
        pltpu.sync_copy(idx_hbm.at[pl.ds(off, GR)], idx_v)
        pltpu.async_copy(tab_hbm.at[idx_v], rows_v, sem).wait()
        pltpu.sync_copy(rows_v, out_hbm.at[pl.ds(off, GR)])
        return 0

    lax.fori_loop(0, (NP // (NCORE * NSUB)) // GR, chunk, 0)


_emb_gather = pl.kernel(
    _emb_body,
    out_type=jax.ShapeDtypeStruct((NP, D), jnp.float32),
    mesh=_mesh,
    scratch_types=[
        pltpu.VMEM((GR,), jnp.int32),
        pltpu.VMEM((GR, D), jnp.float32),
        pltpu.SemaphoreType.DMA,
    ],
)


def _edge_body(src_hbm, dst_hbm, h0_hbm, h1_hbm, a0_hbm, a1_hbm, b0_hbm, b1_hbm,
               znd_hbm, zn_hbm,
               numcat_hbm, dencat_hbm,
               sidxA, didxA, sidxB, didxB,
               avA, bvA, wvA, wpA, rowsA, avB, bvB, wvB, wpB, rowsB,
               acc, den_sh,
               sem_rA, sem_aA, sem_bA, sem_rB, sem_aB, sem_bB, sem_iA, sem_iB):
    c = lax.axis_index("c")
    s = lax.axis_index("s")
    r0 = s * EPS
    coff = c * NP

    pltpu.sync_copy(znd_hbm.at[pl.ds(r0, EPS)], acc.at[pl.ds(r0, EPS)])
    pltpu.sync_copy(zn_hbm.at[pl.ds(r0, EPS)], den_sh.at[pl.ds(r0, EPS)])
    plsc.subcore_barrier()

    ebase = s * (CHUNKS * EC)

    def fetch_idx(ch, sidx_b, didx_b, sem_i):
        off = ebase + ch * EC
        pltpu.async_copy(src_hbm.at[pl.ds(off, EC)], sidx_b, sem_i)
        pltpu.async_copy(dst_hbm.at[pl.ds(off, EC)], didx_b, sem_i)

    def wait_idx(ch, sidx_b, didx_b, sem_i):
        off = ebase + ch * EC
        pltpu.make_async_copy(src_hbm.at[pl.ds(off, EC)], sidx_b, sem_i).wait()
        pltpu.make_async_copy(dst_hbm.at[pl.ds(off, EC)], didx_b, sem_i).wait()

    def issue(sidx_b, didx_b, rows_b, av_b, bv_b, sem_r, sem_a, sem_b):
        @pl.when(c == 0)
        def _():
            pltpu.async_copy(h0_hbm.at[sidx_b], rows_b, sem_r)
            pltpu.async_copy(a0_hbm.at[sidx_b], av_b, sem_a)
            pltpu.async_copy(b0_hbm.at[didx_b], bv_b, sem_b)

        @pl.when(c == 1)
        def _():
            pltpu.async_copy(h1_hbm.at[sidx_b], rows_b, sem_r)
            pltpu.async_copy(a1_hbm.at[sidx_b], av_b, sem_a)
            pltpu.async_copy(b1_hbm.at[didx_b], bv_b, sem_b)

    def process(sidx_b, didx_b, rows_b, av_b, bv_b, wv_b, wp_b, sem_r, sem_a, sem_b):
        pltpu.make_async_copy(a0_hbm.at[sidx_b], av_b, sem_a).wait()
        pltpu.make_async_copy(b0_hbm.at[didx_b], bv_b, sem_b).wait()
        for j in range(EC // 16):
            v = av_b[pl.ds(j * 16, 16)] + bv_b[pl.ds(j * 16, 16)]
            w16 = jnp.exp(_leaky(v, 0.2))
            wv_b[pl.ds(j * 16, 16)] = w16
            wp_b[pl.ds(j * 16, 16)] = w16
        pltpu.make_async_copy(h0_hbm.at[sidx_b], rows_b, sem_r).wait()

        def scale(e4, _):
            for u in range(4):
                e = e4 * 4 + u
                ws = wp_b[pl.ds(e, 16)][0]
                for f in range(D // 16):
                    rows_b[e, pl.ds(f * 16, 16)] = rows_b[e, pl.ds(f * 16, 16)] * ws
            return 0

        lax.fori_loop(0, EC // 4, scale, 0)
        pltpu.sync_copy(rows_b, acc.at[didx_b], add=True)
        pltpu.sync_copy(wv_b, den_sh.at[didx_b], add=True)

    fetch_idx(0, sidxA, didxA, sem_iA)
    wait_idx(0, sidxA, didxA, sem_iA)
    issue(sidxA, didxA, rowsA, avA, bvA, sem_rA, sem_aA, sem_bA)
    fetch_idx(1, sidxB, didxB, sem_iB)

    def body(k, _):
        cha = k * 2
        chb = k * 2 + 1
        wait_idx(chb, sidxB, didxB, sem_iB)
        issue(sidxB, didxB, rowsB, avB, bvB, sem_rB, sem_aB, sem_bB)
        process(sidxA, didxA, rowsA, avA, bvA, wvA, wpA, sem_rA, sem_aA, sem_bA)

        @pl.when(cha + 2 < CHUNKS)
        def _():
            fetch_idx(cha + 2, sidxA, didxA, sem_iA)

        @pl.when(chb + 1 < CHUNKS)
        def _():
            wait_idx(chb + 1, sidxA, didxA, sem_iA)
            issue(sidxA, didxA, rowsA, avA, bvA, sem_rA, sem_aA, sem_bA)

        process(sidxB, didxB, rowsB, avB, bvB, wvB, wpB, sem_rB, sem_aB, sem_bB)

        @pl.when(chb + 2 < CHUNKS)
        def _():
            fetch_idx(chb + 2, sidxB, didxB, sem_iB)

        return 0

    lax.fori_loop(0, CHUNKS // 2, body, 0)
    plsc.subcore_barrier()
    pltpu.sync_copy(acc.at[pl.ds(r0, EPS)], numcat_hbm.at[pl.ds(coff + r0, EPS)])
    pltpu.sync_copy(den_sh.at[pl.ds(r0, EPS)], dencat_hbm.at[pl.ds(coff + r0, EPS)])


_edge_sc = pl.kernel(
    _edge_body,
    out_type=(
        jax.ShapeDtypeStruct((2 * NP, HID), jnp.float32),
        jax.ShapeDtypeStruct((2 * NP,), jnp.float32),
    ),
    mesh=_mesh,
    scratch_types=[
        pltpu.VMEM((EC,), jnp.int32),
        pltpu.VMEM((EC,), jnp.int32),
        pltpu.VMEM((EC,), jnp.int32),
        pltpu.VMEM((EC,), jnp.int32),
        pltpu.VMEM((EC,), jnp.float32),
        pltpu.VMEM((EC,), jnp.float32),
        pltpu.VMEM((EC,), jnp.float32),
        pltpu.VMEM((EC + 16,), jnp.float32),
        pltpu.VMEM((EC, HID), jnp.float32),
        pltpu.VMEM((EC,), jnp.float32),
        pltpu.VMEM((EC,), jnp.float32),
        pltpu.VMEM((EC,), jnp.float32),
        pltpu.VMEM((EC + 16,), jnp.float32),
        pltpu.VMEM((EC, HID), jnp.float32),
        pltpu.VMEM_SHARED((NP, HID), jnp.float32),
        pltpu.VMEM_SHARED((NP,), jnp.float32),
        pltpu.SemaphoreType.DMA,
        pltpu.SemaphoreType.DMA,
        pltpu.SemaphoreType.DMA,
        pltpu.SemaphoreType.DMA,
        pltpu.SemaphoreType.DMA,
        pltpu.SemaphoreType.DMA,
        pltpu.SemaphoreType.DMA,
        pltpu.SemaphoreType.DMA,
    ],
)


def _tca_body(xe_ref, g_ref, b_ref, w_ref, as_ref, ad_ref,
              h0_ref, h1_ref, av0_ref, av1_ref, bv0_ref, bv1_ref):
    xb = xe_ref[...]
    mu = jnp.mean(xb, axis=1, keepdims=True)
    var = jnp.mean((xb - mu) ** 2, axis=1, keepdims=True)
    ln = (xb - mu) * lax.rsqrt(var + 1e-5) * g_ref[...] + b_ref[...]
    h = jnp.dot(ln, w_ref[...], preferred_element_type=jnp.float32)
    h0 = h[:, :HID]
    h1 = h[:, HID:]
    h0_ref[...] = h0
    h1_ref[...] = h1
    av0_ref[...] = jnp.sum(h0 * as_ref[0:1, :], axis=1, keepdims=True)
    av1_ref[...] = jnp.sum(h1 * as_ref[1:2, :], axis=1, keepdims=True)
    bv0_ref[...] = jnp.sum(h0 * ad_ref[0:1, :], axis=1, keepdims=True)
    bv1_ref[...] = jnp.sum(h1 * ad_ref[1:2, :], axis=1, keepdims=True)


_TCA_BN = 256


def _tc_a(xe, g, b, W, a_s, a_d):
    grid = (NP // _TCA_BN,)
    row = lambda i: (i, 0)
    fixed = lambda i: (0, 0)
    return pl.pallas_call(
        _tca_body,
        grid=grid,
        in_specs=[
            pl.BlockSpec((_TCA_BN, D), row),
            pl.BlockSpec((1, D), fixed),
            pl.BlockSpec((1, D), fixed),
            pl.BlockSpec((D, HEADS * HID), fixed),
            pl.BlockSpec((HEADS, HID), fixed),
            pl.BlockSpec((HEADS, HID), fixed),
        ],
        out_specs=[
            pl.BlockSpec((_TCA_BN, HID), row),
            pl.BlockSpec((_TCA_BN, HID), row),
            pl.BlockSpec((_TCA_BN, 1), row),
            pl.BlockSpec((_TCA_BN, 1), row),
            pl.BlockSpec((_TCA_BN, 1), row),
            pl.BlockSpec((_TCA_BN, 1), row),
        ],
        out_shape=[
            jax.ShapeDtypeStruct((NP, HID), jnp.float32),
            jax.ShapeDtypeStruct((NP, HID), jnp.float32),
            jax.ShapeDtypeStruct((NP, 1), jnp.float32),
            jax.ShapeDtypeStruct((NP, 1), jnp.float32),
            jax.ShapeDtypeStruct((NP, 1), jnp.float32),
            jax.ShapeDtypeStruct((NP, 1), jnp.float32),
        ],
    )(xe, g.reshape(1, D), b.reshape(1, D), W, a_s.reshape(HEADS, HID),
      a_d.reshape(HEADS, HID))


def _tcb_body(n0_ref, n1_ref, d0_ref, d1_ref, h0_ref, h1_ref,
              av0_ref, av1_ref, bv0_ref, bv1_ref, b1_ref, w2_ref, as_ref, ad_ref,
              o0_ref, o1_ref, av20_ref, av21_ref, bv20_ref, bv21_ref):
    ws0 = jnp.exp(_leaky(av0_ref[...] + bv0_ref[...], 0.2))
    ws1 = jnp.exp(_leaky(av1_ref[...] + bv1_ref[...], 0.2))
    o0 = (n0_ref[...] + ws0 * h0_ref[...]) / (d0_ref[...] + ws0 + 1e-16)
    o1 = (n1_ref[...] + ws1 * h1_ref[...]) / (d1_ref[...] + ws1 + 1e-16)
    gcat = jnp.concatenate([o0, o1], axis=1) + b1_ref[...]
    gcat = _leaky(gcat, 0.05)
    h2 = jnp.dot(gcat, w2_ref[...], preferred_element_type=jnp.float32)
    h20 = h2[:, :HID]
    h21 = h2[:, HID:]
    o0_ref[...] = h20
    o1_ref[...] = h21
    av20_ref[...] = jnp.sum(h20 * as_ref[0:1, :], axis=1, keepdims=True)
    av21_ref[...] = jnp.sum(h21 * as_ref[1:2, :], axis=1, keepdims=True)
    bv20_ref[...] = jnp.sum(h20 * ad_ref[0:1, :], axis=1, keepdims=True)
    bv21_ref[...] = jnp.sum(h21 * ad_ref[1:2, :], axis=1, keepdims=True)


def _tc_b(n0, n1, d0, d1, h0, h1, av0, av1, bv0, bv1, b1, W2, a_s, a_d):
    grid = (NP // _TCA_BN,)
    row = lambda i: (i, 0)
    fixed = lambda i: (0, 0)
    col = pl.BlockSpec((_TCA_BN, 1), row)
    mat = pl.BlockSpec((_TCA_BN, HID), row)
    return pl.pallas_call(
        _tcb_body,
        grid=grid,
        in_specs=[mat, mat, col, col, mat, mat, col, col, col, col,
                  pl.BlockSpec((1, HEADS * HID), fixed),
                  pl.BlockSpec((HEADS * HID, HEADS * HID), fixed),
                  pl.BlockSpec((HEADS, HID), fixed),
                  pl.BlockSpec((HEADS, HID), fixed)],
        out_specs=[mat, mat, col, col, col, col],
        out_shape=[
            jax.ShapeDtypeStruct((NP, HID), jnp.float32),
            jax.ShapeDtypeStruct((NP, HID), jnp.float32),
            jax.ShapeDtypeStruct((NP, 1), jnp.float32),
            jax.ShapeDtypeStruct((NP, 1), jnp.float32),
            jax.ShapeDtypeStruct((NP, 1), jnp.float32),
            jax.ShapeDtypeStruct((NP, 1), jnp.float32),
        ],
    )(n0, n1, d0, d1, h0, h1, av0, av1, bv0, bv1,
      b1.reshape(1, HEADS * HID), W2, a_s.reshape(HEADS, HID), a_d.reshape(HEADS, HID))


_TCC_BN = 200
_TCC_STEPS = N // _TCC_BN


def _tcc_body(n0_ref, n1_ref, d0_ref, d1_ref, h0_ref, h1_ref,
              av0_ref, av1_ref, bv0_ref, bv1_ref, bidx_ref,
              b2_ref, gw1_ref, gb1_ref, gw2_ref, gb2_ref,
              ln2g_ref, ln2b_ref, fw1_ref, fb1_ref, fw2_ref, fb2_ref,
              oc_ref, hid_ref, s_acc):
    i = pl.program_id(0)

    ws0 = jnp.exp(_leaky(av0_ref[...] + bv0_ref[...], 0.2))
    ws1 = jnp.exp(_leaky(av1_ref[...] + bv1_ref[...], 0.2))
    o0 = (n0_ref[...] + ws0 * h0_ref[...]) / (d0_ref[...] + ws0 + 1e-16)
    o1 = (n1_ref[...] + ws1 * h1_ref[...]) / (d1_ref[...] + ws1 + 1e-16)
    oc = jnp.concatenate([o0, o1], axis=1) + b2_ref[...]
    oc_ref[...] = oc

    gate = _leaky(jnp.dot(oc, gw1_ref[...], preferred_element_type=jnp.float32)
                  + gb1_ref[...], 0.05)
    gate = jnp.dot(gate, gw2_ref[...], preferred_element_type=jnp.float32) + gb2_ref[...]
    e = jnp.exp(gate)

    ids = lax.broadcasted_iota(jnp.int32, (1, B), 1)
    m = (bidx_ref[...] == ids).astype(jnp.float32)
    xcat = jnp.concatenate([e * oc, jnp.broadcast_to(e, (_TCC_BN, HID))], axis=1)
    part = lax.dot_general(m, xcat, (((0,), (0,)), ((), ())),
                           preferred_element_type=jnp.float32)

    @pl.when(i == 0)
    def _():
        s_acc[...] = jnp.zeros_like(s_acc)

    s_acc[...] += part

    @pl.when(i == _TCC_STEPS - 1)
    def _():
        s = s_acc[...]
        hidden = s[:, :HEADS * HID] / (s[:, HEADS * HID:HEADS * HID + 1] + 1e-16)
        mu = jnp.mean(hidden, axis=1, keepdims=True)
        var = jnp.mean((hidden - mu) ** 2, axis=1, keepdims=True)
        hidden = (hidden - mu) * lax.rsqrt(var + 1e-5) * ln2g_ref[...] + ln2b_ref[...]
        hidden = _leaky(jnp.dot(hidden, fw1_ref[...],
                                preferred_element_type=jnp.float32) + fb1_ref[...], 0.05)
        hid_ref[...] = jnp.dot(hidden, fw2_ref[...],
                               preferred_element_type=jnp.float32) + fb2_ref[...]


def _tc_c(n0, n1, d0, d1, h0, h1, av0, av1, bv0, bv1, bidx,
          b2, gW1, gb1, gW2, gb2, ln2g, ln2b, fW1, fb1, fW2, fb2):
    grid = (_TCC_STEPS,)
    row = lambda i: (i, 0)
    fixed = lambda i: (0, 0)
    col = pl.BlockSpec((_TCC_BN, 1), row)
    mat = pl.BlockSpec((_TCC_BN, HID), row)
    return pl.pallas_call(
        _tcc_body,
        grid=grid,
        in_specs=[mat, mat, col, col, mat, mat, col, col, col, col,
                  pl.BlockSpec((_TCC_BN, 1), row),
                  pl.BlockSpec((1, HEADS * HID), fixed),
                  pl.BlockSpec((HEADS * HID, HID), fixed),
                  pl.BlockSpec((1, HID), fixed),
                  pl.BlockSpec((HID, 1), fixed),
                  pl.BlockSpec((1, 1), fixed),
                  pl.BlockSpec((1, HEADS * HID), fixed),
                  pl.BlockSpec((1, HEADS * HID), fixed),
                  pl.BlockSpec((HEADS * HID, HEADS * HID), fixed),
                  pl.BlockSpec((1, HEADS * HID), fixed),
                  pl.BlockSpec((HEADS * HID, HID), fixed),
                  pl.BlockSpec((1, HID), fixed)],
        out_specs=[
            pl.BlockSpec((_TCC_BN, HEADS * HID), row),
            pl.BlockSpec((B, HID), fixed),
        ],
        out_shape=[
            jax.ShapeDtypeStruct((N, HEADS * HID), jnp.float32),
            jax.ShapeDtypeStruct((B, HID), jnp.float32),
        ],
        scratch_shapes=[pltpu.VMEM((B, HEADS * HID + HID), jnp.float32)],
    )(n0, n1, d0, d1, h0, h1, av0, av1, bv0, bv1, bidx,
      b2.reshape(1, HEADS * HID), gW1, gb1.reshape(1, HID), gW2,
      gb2.reshape(1, 1), ln2g.reshape(1, HEADS * HID), ln2b.reshape(1, HEADS * HID),
      fW1, fb1.reshape(1, HEADS * HID), fW2, fb2.reshape(1, HID))


def kernel(x, edge_index, batch_idx, emb_table, ln1_g, ln1_b, W1, a_src1, a_dst1, b1,
           W2, a_src2, a_dst2, b2, gW1, gb1, gW2, gb2, ln2_g, ln2_b, fW1, fb1, fW2, fb2):
    pad = E_PAD - E
    src = jnp.concatenate([edge_index[0].astype(jnp.int32),
                           jnp.zeros((pad,), jnp.int32)])
    dst = jnp.concatenate([edge_index[1].astype(jnp.int32),
                           jnp.full((pad,), NP - 1, jnp.int32)])
    xp = jnp.concatenate([x[:, 0].astype(jnp.int32), jnp.zeros((NP - N,), jnp.int32)])
    znd = jnp.zeros((NP, HID), jnp.float32)
    zn = jnp.zeros((NP,), jnp.float32)

    xe = _emb_gather(emb_table, xp)
    h0, h1, av0, av1, bv0, bv1 = _tc_a(xe, ln1_g, ln1_b, W1, a_src1, a_dst1)
    ncat, dcat = _edge_sc(src, dst, h0, h1,
                          av0.reshape(NP), av1.reshape(NP),
                          bv0.reshape(NP), bv1.reshape(NP), znd, zn)
    h20, h21, av20, av21, bv20, bv21 = _tc_b(
        ncat[:NP], ncat[NP:], dcat[:NP].reshape(NP, 1), dcat[NP:].reshape(NP, 1),
        h0, h1, av0, av1, bv0, bv1, b1, W2, a_src2, a_dst2)
    ncat, dcat = _edge_sc(src, dst, h20, h21,
                          av20.reshape(NP), av21.reshape(NP),
                          bv20.reshape(NP), bv21.reshape(NP), znd, zn)
    out_conv, hidden = _tc_c(
        ncat[:NP], ncat[NP:], dcat[:NP].reshape(NP, 1), dcat[NP:].reshape(NP, 1),
        h20, h21, av20, av21, bv20, bv21, batch_idx.astype(jnp.int32).reshape(N, 1),
        b2, gW1, gb1, gW2, gb2, ln2_g, ln2_b, fW1, fb1, fW2, fb2)
    return out_conv, hidden


# speedup vs baseline: 36.3396x; 1.0064x over previous
import functools

import jax
import jax.numpy as jnp
from jax import lax
from jax.experimental import pallas as pl
from jax.experimental.pallas import tpu as pltpu
from jax.experimental.pallas import tpu_sc as plsc

N = 10000
E = 320000
V = 5001
D = 128
HID = 128
HEADS = 2
B = 64

NP = 10240
NSUB = 16
NCORE = 2
EC = 128
CHUNKS = 160
EPS = NP // NSUB
E_PAD = NSUB * CHUNKS * EC
GR = 320 // 4

_mesh = plsc.VectorSubcoreMesh(
    core_axis_name="c", subcore_axis_name="s", num_cores=NCORE, num_subcores=NSUB)


def _leaky(x, s):
    return jnp.where(x > 0, x, s * x)


def _emb_body(tab_hbm, idx_hbm, out_hbm, idx_v, rows_v, sem):
    c = lax.axis_index("c")
    s = lax.axis_index("s")
    wid = s * NCORE + c
    base = wid * (NP // (NCORE * NSUB))

    def chunk(i, _):
        off = base + i * GR
        pltpu.sync_copy(idx_hbm.at[pl.ds(off, GR)], idx_v)
        pltpu.async_copy(tab_hbm.at[idx_v], rows_v, sem).wait()
        pltpu.sync_copy(rows_v, out_hbm.at[pl.ds(off, GR)])
        return 0

    lax.fori_loop(0, (NP // (NCORE * NSUB)) // GR, chunk, 0)


_emb_gather = pl.kernel(
    _emb_body,
    out_type=jax.ShapeDtypeStruct((NP, D), jnp.float32),
    mesh=_mesh,
    scratch_types=[
        pltpu.VMEM((GR,), jnp.int32),
        pltpu.VMEM((GR, D), jnp.float32),
        pltpu.SemaphoreType.DMA,
    ],
)


def _edge_body(src_hbm, dst_hbm, h0_hbm, h1_hbm, a0_hbm, a1_hbm, b0_hbm, b1_hbm,
               znd_hbm, zn_hbm,
               numcat_hbm, dencat_hbm,
               sidxA, didxA, sidxB, didxB,
               avA, bvA, wvA, wpA, rowsA, avB, bvB, wvB, wpB, rowsB,
               acc, den_sh,
               sem_rA, sem_aA, sem_bA, sem_rB, sem_aB, sem_bB, sem_iA, sem_iB,
               sem_dA, sem_dB):
    c = lax.axis_index("c")
    s = lax.axis_index("s")
    r0 = s * EPS
    coff = c * NP

    pltpu.sync_copy(znd_hbm.at[pl.ds(r0, EPS)], acc.at[pl.ds(r0, EPS)])
    pltpu.sync_copy(zn_hbm.at[pl.ds(r0, EPS)], den_sh.at[pl.ds(r0, EPS)])
    plsc.subcore_barrier()

    ebase = s * (CHUNKS * EC)

    def fetch_idx(ch, sidx_b, didx_b, sem_i):
        off = ebase + ch * EC
        pltpu.async_copy(src_hbm.at[pl.ds(off, EC)], sidx_b, sem_i)
        pltpu.async_copy(dst_hbm.at[pl.ds(off, EC)], didx_b, sem_i)

    def wait_idx(ch, sidx_b, didx_b, sem_i):
        off = ebase + ch * EC
        pltpu.make_async_copy(src_hbm.at[pl.ds(off, EC)], sidx_b, sem_i).wait()
        pltpu.make_async_copy(dst_hbm.at[pl.ds(off, EC)], didx_b, sem_i).wait()

    def issue(sidx_b, didx_b, rows_b, av_b, bv_b, sem_r, sem_a, sem_b):
        @pl.when(c == 0)
        def _():
            pltpu.async_copy(h0_hbm.at[sidx_b], rows_b, sem_r)
            pltpu.async_copy(a0_hbm.at[sidx_b], av_b, sem_a)
            pltpu.async_copy(b0_hbm.at[didx_b], bv_b, sem_b)

        @pl.when(c == 1)
        def _():
            pltpu.async_copy(h1_hbm.at[sidx_b], rows_b, sem_r)
            pltpu.async_copy(a1_hbm.at[sidx_b], av_b, sem_a)
            pltpu.async_copy(b1_hbm.at[didx_b], bv_b, sem_b)

    def den_wait(didx_b, wv_b, sem_d):
        pltpu.make_async_copy(wv_b, den_sh.at[didx_b], sem_d).wait()

    def process(sidx_b, didx_b, rows_b, av_b, bv_b, wv_b, wp_b, sem_r, sem_a, sem_b,
                sem_d):
        pltpu.make_async_copy(a0_hbm.at[sidx_b], av_b, sem_a).wait()
        pltpu.make_async_copy(b0_hbm.at[didx_b], bv_b, sem_b).wait()
        for j in range(EC // 16):
            v = av_b[pl.ds(j * 16, 16)] + bv_b[pl.ds(j * 16, 16)]
            w16 = jnp.exp(_leaky(v, 0.2))
            wv_b[pl.ds(j * 16, 16)] = w16
            wp_b[pl.ds(j * 16, 16)] = w16
        pltpu.make_async_copy(h0_hbm.at[sidx_b], rows_b, sem_r).wait()

        def scale(e4, _):
            for u in range(4):
                e = e4 * 4 + u
                ws = wp_b[pl.ds(e, 16)][0]
                for f in range(D // 16):
                    rows_b[e, pl.ds(f * 16, 16)] = rows_b[e, pl.ds(f * 16, 16)] * ws
            return 0

        lax.fori_loop(0, EC // 4, scale, 0)
        pltpu.async_copy(wv_b, den_sh.at[didx_b], sem_d, add=True)
        pltpu.sync_copy(rows_b, acc.at[didx_b], add=True)

    fetch_idx(0, sidxA, didxA, sem_iA)
    wait_idx(0, sidxA, didxA, sem_iA)
    issue(sidxA, didxA, rowsA, avA, bvA, sem_rA, sem_aA, sem_bA)
    fetch_idx(1, sidxB, didxB, sem_iB)

    def body(k, _):
        cha = k * 2
        chb = k * 2 + 1
        wait_idx(chb, sidxB, didxB, sem_iB)
        issue(sidxB, didxB, rowsB, avB, bvB, sem_rB, sem_aB, sem_bB)

        process(sidxA, didxA, rowsA, avA, bvA, wvA, wpA, sem_rA, sem_aA, sem_bA, sem_dA)

        @pl.when(cha + 2 < CHUNKS)
        def _():
            den_wait(didxA, wvA, sem_dA)
            fetch_idx(cha + 2, sidxA, didxA, sem_iA)

        @pl.when(chb + 1 < CHUNKS)
        def _():
            wait_idx(chb + 1, sidxA, didxA, sem_iA)
            issue(sidxA, didxA, rowsA, avA, bvA, sem_rA, sem_aA, sem_bA)

        process(sidxB, didxB, rowsB, avB, bvB, wvB, wpB, sem_rB, sem_aB, sem_bB, sem_dB)

        @pl.when(chb + 2 < CHUNKS)
        def _():
            den_wait(didxB, wvB, sem_dB)
            fetch_idx(chb + 2, sidxB, didxB, sem_iB)

        return 0

    lax.fori_loop(0, CHUNKS // 2, body, 0)
    den_wait(didxA, wvA, sem_dA)
    den_wait(didxB, wvB, sem_dB)
    plsc.subcore_barrier()
    pltpu.sync_copy(acc.at[pl.ds(r0, EPS)], numcat_hbm.at[pl.ds(coff + r0, EPS)])
    pltpu.sync_copy(den_sh.at[pl.ds(r0, EPS)], dencat_hbm.at[pl.ds(coff + r0, EPS)])


_edge_sc = pl.kernel(
    _edge_body,
    out_type=(
        jax.ShapeDtypeStruct((2 * NP, HID), jnp.float32),
        jax.ShapeDtypeStruct((2 * NP,), jnp.float32),
    ),
    mesh=_mesh,
    scratch_types=[
        pltpu.VMEM((EC,), jnp.int32),
        pltpu.VMEM((EC,), jnp.int32),
        pltpu.VMEM((EC,), jnp.int32),
        pltpu.VMEM((EC,), jnp.int32),
        pltpu.VMEM((EC,), jnp.float32),
        pltpu.VMEM((EC,), jnp.float32),
        pltpu.VMEM((EC,), jnp.float32),
        pltpu.VMEM((EC + 16,), jnp.float32),
        pltpu.VMEM((EC, HID), jnp.float32),
        pltpu.VMEM((EC,), jnp.float32),
        pltpu.VMEM((EC,), jnp.float32),
        pltpu.VMEM((EC,), jnp.float32),
        pltpu.VMEM((EC + 16,), jnp.float32),
        pltpu.VMEM((EC, HID), jnp.float32),
        pltpu.VMEM_SHARED((NP, HID), jnp.float32),
        pltpu.VMEM_SHARED((NP,), jnp.float32),
        pltpu.SemaphoreType.DMA,
        pltpu.SemaphoreType.DMA,
        pltpu.SemaphoreType.DMA,
        pltpu.SemaphoreType.DMA,
        pltpu.SemaphoreType.DMA,
        pltpu.SemaphoreType.DMA,
        pltpu.SemaphoreType.DMA,
        pltpu.SemaphoreType.DMA,
        pltpu.SemaphoreType.DMA,
        pltpu.SemaphoreType.DMA,
    ],
)


def _tca_body(xe_ref, g_ref, b_ref, w_ref, as_ref, ad_ref,
              h0_ref, h1_ref, av0_ref, av1_ref, bv0_ref, bv1_ref):
    xb = xe_ref[...]
    mu = jnp.mean(xb, axis=1, keepdims=True)
    var = jnp.mean((xb - mu) ** 2, axis=1, keepdims=True)
    ln = (xb - mu) * lax.rsqrt(var + 1e-5) * g_ref[...] + b_ref[...]
    h = jnp.dot(ln, w_ref[...], preferred_element_type=jnp.float32)
    h0 = h[:, :HID]
    h1 = h[:, HID:]
    h0_ref[...] = h0
    h1_ref[...] = h1
    av0_ref[...] = jnp.sum(h0 * as_ref[0:1, :], axis=1, keepdims=True)
    av1_ref[...] = jnp.sum(h1 * as_ref[1:2, :], axis=1, keepdims=True)
    bv0_ref[...] = jnp.sum(h0 * ad_ref[0:1, :], axis=1, keepdims=True)
    bv1_ref[...] = jnp.sum(h1 * ad_ref[1:2, :], axis=1, keepdims=True)


_TCA_BN = 256


def _tc_a(xe, g, b, W, a_s, a_d):
    grid = (NP // _TCA_BN,)
    row = lambda i: (i, 0)
    fixed = lambda i: (0, 0)
    return pl.pallas_call(
        _tca_body,
        grid=grid,
        in_specs=[
            pl.BlockSpec((_TCA_BN, D), row),
            pl.BlockSpec((1, D), fixed),
            pl.BlockSpec((1, D), fixed),
            pl.BlockSpec((D, HEADS * HID), fixed),
            pl.BlockSpec((HEADS, HID), fixed),
            pl.BlockSpec((HEADS, HID), fixed),
        ],
        out_specs=[
            pl.BlockSpec((_TCA_BN, HID), row),
            pl.BlockSpec((_TCA_BN, HID), row),
            pl.BlockSpec((_TCA_BN, 1), row),
            pl.BlockSpec((_TCA_BN, 1), row),
            pl.BlockSpec((_TCA_BN, 1), row),
            pl.BlockSpec((_TCA_BN, 1), row),
        ],
        out_shape=[
            jax.ShapeDtypeStruct((NP, HID), jnp.float32),
            jax.ShapeDtypeStruct((NP, HID), jnp.float32),
            jax.ShapeDtypeStruct((NP, 1), jnp.float32),
            jax.ShapeDtypeStruct((NP, 1), jnp.float32),
            jax.ShapeDtypeStruct((NP, 1), jnp.float32),
            jax.ShapeDtypeStruct((NP, 1), jnp.float32),
        ],
    )(xe, g.reshape(1, D), b.reshape(1, D), W, a_s.reshape(HEADS, HID),
      a_d.reshape(HEADS, HID))


def _tcb_body(n0_ref, n1_ref, d0_ref, d1_ref, h0_ref, h1_ref,
              av0_ref, av1_ref, bv0_ref, bv1_ref, b1_ref, w2_ref, as_ref, ad_ref,
              o0_ref, o1_ref, av20_ref, av21_ref, bv20_ref, bv21_ref):
    ws0 = jnp.exp(_leaky(av0_ref[...] + bv0_ref[...], 0.2))
    ws1 = jnp.exp(_leaky(av1_ref[...] + bv1_ref[...], 0.2))
    o0 = (n0_ref[...] + ws0 * h0_ref[...]) / (d0_ref[...] + ws0 + 1e-16)
    o1 = (n1_ref[...] + ws1 * h1_ref[...]) / (d1_ref[...] + ws1 + 1e-16)
    gcat = jnp.concatenate([o0, o1], axis=1) + b1_ref[...]
    gcat = _leaky(gcat, 0.05)
    h2 = jnp.dot(gcat, w2_ref[...], preferred_element_type=jnp.float32)
    h20 = h2[:, :HID]
    h21 = h2[:, HID:]
    o0_ref[...] = h20
    o1_ref[...] = h21
    av20_ref[...] = jnp.sum(h20 * as_ref[0:1, :], axis=1, keepdims=True)
    av21_ref[...] = jnp.sum(h21 * as_ref[1:2, :], axis=1, keepdims=True)
    bv20_ref[...] = jnp.sum(h20 * ad_ref[0:1, :], axis=1, keepdims=True)
    bv21_ref[...] = jnp.sum(h21 * ad_ref[1:2, :], axis=1, keepdims=True)


def _tc_b(n0, n1, d0, d1, h0, h1, av0, av1, bv0, bv1, b1, W2, a_s, a_d):
    grid = (NP // _TCA_BN,)
    row = lambda i: (i, 0)
    fixed = lambda i: (0, 0)
    col = pl.BlockSpec((_TCA_BN, 1), row)
    mat = pl.BlockSpec((_TCA_BN, HID), row)
    return pl.pallas_call(
        _tcb_body,
        grid=grid,
        in_specs=[mat, mat, col, col, mat, mat, col, col, col, col,
                  pl.BlockSpec((1, HEADS * HID), fixed),
                  pl.BlockSpec((HEADS * HID, HEADS * HID), fixed),
                  pl.BlockSpec((HEADS, HID), fixed),
                  pl.BlockSpec((HEADS, HID), fixed)],
        out_specs=[mat, mat, col, col, col, col],
        out_shape=[
            jax.ShapeDtypeStruct((NP, HID), jnp.float32),
            jax.ShapeDtypeStruct((NP, HID), jnp.float32),
            jax.ShapeDtypeStruct((NP, 1), jnp.float32),
            jax.ShapeDtypeStruct((NP, 1), jnp.float32),
            jax.ShapeDtypeStruct((NP, 1), jnp.float32),
            jax.ShapeDtypeStruct((NP, 1), jnp.float32),
        ],
    )(n0, n1, d0, d1, h0, h1, av0, av1, bv0, bv1,
      b1.reshape(1, HEADS * HID), W2, a_s.reshape(HEADS, HID), a_d.reshape(HEADS, HID))


_TCC_BN = 200
_TCC_STEPS = N // _TCC_BN


def _tcc_body(n0_ref, n1_ref, d0_ref, d1_ref, h0_ref, h1_ref,
              av0_ref, av1_ref, bv0_ref, bv1_ref, bidx_ref,
              b2_ref, gw1_ref, gb1_ref, gw2_ref, gb2_ref,
              ln2g_ref, ln2b_ref, fw1_ref, fb1_ref, fw2_ref, fb2_ref,
              oc_ref, hid_ref, s_acc):
    i = pl.program_id(0)

    ws0 = jnp.exp(_leaky(av0_ref[...] + bv0_ref[...], 0.2))
    ws1 = jnp.exp(_leaky(av1_ref[...] + bv1_ref[...], 0.2))
    o0 = (n0_ref[...] + ws0 * h0_ref[...]) / (d0_ref[...] + ws0 + 1e-16)
    o1 = (n1_ref[...] + ws1 * h1_ref[...]) / (d1_ref[...] + ws1 + 1e-16)
    oc = jnp.concatenate([o0, o1], axis=1) + b2_ref[...]
    oc_ref[...] = oc

    gate = _leaky(jnp.dot(oc, gw1_ref[...], preferred_element_type=jnp.float32)
                  + gb1_ref[...], 0.05)
    gate = jnp.dot(gate, gw2_ref[...], preferred_element_type=jnp.float32) + gb2_ref[...]
    e = jnp.exp(gate)

    ids = lax.broadcasted_iota(jnp.int32, (1, B), 1)
    m = (bidx_ref[...] == ids).astype(jnp.float32)
    xcat = jnp.concatenate([e * oc, jnp.broadcast_to(e, (_TCC_BN, HID))], axis=1)
    part = lax.dot_general(m, xcat, (((0,), (0,)), ((), ())),
                           preferred_element_type=jnp.float32)

    @pl.when(i == 0)
    def _():
        s_acc[...] = jnp.zeros_like(s_acc)

    s_acc[...] += part

    @pl.when(i == _TCC_STEPS - 1)
    def _():
        s = s_acc[...]
        hidden = s[:, :HEADS * HID] / (s[:, HEADS * HID:HEADS * HID + 1] + 1e-16)
        mu = jnp.mean(hidden, axis=1, keepdims=True)
        var = jnp.mean((hidden - mu) ** 2, axis=1, keepdims=True)
        hidden = (hidden - mu) * lax.rsqrt(var + 1e-5) * ln2g_ref[...] + ln2b_ref[...]
        hidden = _leaky(jnp.dot(hidden, fw1_ref[...],
                                preferred_element_type=jnp.float32) + fb1_ref[...], 0.05)
        hid_ref[...] = jnp.dot(hidden, fw2_ref[...],
                               preferred_element_type=jnp.float32) + fb2_ref[...]


def _tc_c(n0, n1, d0, d1, h0, h1, av0, av1, bv0, bv1, bidx,
          b2, gW1, gb1, gW2, gb2, ln2g, ln2b, fW1, fb1, fW2, fb2):
    grid = (_TCC_STEPS,)
    row = lambda i: (i, 0)
    fixed = lambda i: (0, 0)
    col = pl.BlockSpec((_TCC_BN, 1), row)
    mat = pl.BlockSpec((_TCC_BN, HID), row)
    return pl.pallas_call(
        _tcc_body,
        grid=grid,
        in_specs=[mat, mat, col, col, mat, mat, col, col, col, col,
                  pl.BlockSpec((_TCC_BN, 1), row),
                  pl.BlockSpec((1, HEADS * HID), fixed),
                  pl.BlockSpec((HEADS * HID, HID), fixed),
                  pl.BlockSpec((1, HID), fixed),
                  pl.BlockSpec((HID, 1), fixed),
                  pl.BlockSpec((1, 1), fixed),
                  pl.BlockSpec((1, HEADS * HID), fixed),
                  pl.BlockSpec((1, HEADS * HID), fixed),
                  pl.BlockSpec((HEADS * HID, HEADS * HID), fixed),
                  pl.BlockSpec((1, HEADS * HID), fixed),
                  pl.BlockSpec((HEADS * HID, HID), fixed),
                  pl.BlockSpec((1, HID), fixed)],
        out_specs=[
            pl.BlockSpec((_TCC_BN, HEADS * HID), row),
            pl.BlockSpec((B, HID), fixed),
        ],
        out_shape=[
            jax.ShapeDtypeStruct((N, HEADS * HID), jnp.float32),
            jax.ShapeDtypeStruct((B, HID), jnp.float32),
        ],
        scratch_shapes=[pltpu.VMEM((B, HEADS * HID + HID), jnp.float32)],
    )(n0, n1, d0, d1, h0, h1, av0, av1, bv0, bv1, bidx,
      b2.reshape(1, HEADS * HID), gW1, gb1.reshape(1, HID), gW2,
      gb2.reshape(1, 1), ln2g.reshape(1, HEADS * HID), ln2b.reshape(1, HEADS * HID),
      fW1, fb1.reshape(1, HEADS * HID), fW2, fb2.reshape(1, HID))


def kernel(x, edge_index, batch_idx, emb_table, ln1_g, ln1_b, W1, a_src1, a_dst1, b1,
           W2, a_src2, a_dst2, b2, gW1, gb1, gW2, gb2, ln2_g, ln2_b, fW1, fb1, fW2, fb2):
    pad = E_PAD - E
    src = jnp.concatenate([edge_index[0].astype(jnp.int32),
                           jnp.zeros((pad,), jnp.int32)])
    dst = jnp.concatenate([edge_index[1].astype(jnp.int32),
                           jnp.full((pad,), NP - 1, jnp.int32)])
    xp = jnp.concatenate([x[:, 0].astype(jnp.int32), jnp.zeros((NP - N,), jnp.int32)])
    znd = jnp.zeros((NP, HID), jnp.float32)
    zn = jnp.zeros((NP,), jnp.float32)

    xe = _emb_gather(emb_table, xp)
    h0, h1, av0, av1, bv0, bv1 = _tc_a(xe, ln1_g, ln1_b, W1, a_src1, a_dst1)
    ncat, dcat = _edge_sc(src, dst, h0, h1,
                          av0.reshape(NP), av1.reshape(NP),
                          bv0.reshape(NP), bv1.reshape(NP), znd, zn)
    h20, h21, av20, av21, bv20, bv21 = _tc_b(
        ncat[:NP], ncat[NP:], dcat[:NP].reshape(NP, 1), dcat[NP:].reshape(NP, 1),
        h0, h1, av0, av1, bv0, bv1, b1, W2, a_src2, a_dst2)
    ncat, dcat = _edge_sc(src, dst, h20, h21,
                          av20.reshape(NP), av21.reshape(NP),
                          bv20.reshape(NP), bv21.reshape(NP), znd, zn)
    out_conv, hidden = _tc_c(
        ncat[:NP], ncat[NP:], dcat[:NP].reshape(NP, 1), dcat[NP:].reshape(NP, 1),
        h20, h21, av20, av21, bv20, bv21, batch_idx.astype(jnp.int32).reshape(N, 1),
        b2, gW1, gb1, gW2, gb2, ln2_g, ln2_b, fW1, fb1, fW2, fb2)
    return out_conv, hidden
